# baseline (speedup 1.0000x reference)
import dataclasses
import functools

import jax
import jax.numpy as jnp
import numpy as np
from jax import lax
from jax.experimental import pallas as pl
from jax.experimental.pallas import tpu as pltpu
from jax.experimental.pallas import tpu_sc as plsc

N = 320000
IN_C = 128
OUT_C = 128
NUM_SEG = 10000
EPS = 0.001

NLANE = 16
NVEC = OUT_C // NLANE

_INFO = plsc.get_sparse_core_info()
NW = _INFO.num_cores * _INFO.num_subcores
SPW = 320
CH = 128
BR = 6400


HC = OUT_C // 2


def _mm_stats_body(in_ref, w1et_ref, w1ot_ref, g1_ref, b1_ref, x_ref, ab_ref,
                   s_acc, sq_acc):
    xe = jnp.dot(in_ref[...], w1et_ref[...], preferred_element_type=jnp.float32)
    xo = jnp.dot(in_ref[...], w1ot_ref[...], preferred_element_type=jnp.float32)
    ie = lax.bitcast_convert_type(xe, jnp.int32) + jnp.int32(0x8000)
    io = lax.bitcast_convert_type(xo, jnp.int32) + jnp.int32(0x8000)
    lo = lax.shift_right_logical(ie, 16)
    hi = io & jnp.int32(-65536)
    x_ref[...] = hi | lo

    @pl.when(pl.program_id(0) == 0)
    def _():
        s_acc[...] = jnp.zeros_like(s_acc)
        sq_acc[...] = jnp.zeros_like(sq_acc)

    s_acc[...] += jnp.concatenate(
        [jnp.sum(xe, axis=0, keepdims=True),
         jnp.sum(xo, axis=0, keepdims=True)], axis=0)
    sq_acc[...] += jnp.concatenate(
        [jnp.sum(xe * xe, axis=0, keepdims=True),
         jnp.sum(xo * xo, axis=0, keepdims=True)], axis=0)

    @pl.when(pl.program_id(0) == pl.num_programs(0) - 1)
    def _():
        mu = s_acc[...] / N
        var = sq_acc[...] / N - mu * mu
        a = g1_ref[...] * lax.rsqrt(var + EPS)
        b = b1_ref[...] - a * mu
        ab_ref[...] = jnp.concatenate([a, b], axis=0)


_phase_a = pl.pallas_call(
    _mm_stats_body,
    grid=(N // BR,),
    in_specs=[
        pl.BlockSpec((BR, IN_C), lambda i: (i, 0)),
        pl.BlockSpec((IN_C, HC), lambda i: (0, 0)),
        pl.BlockSpec((IN_C, HC), lambda i: (0, 0)),
        pl.BlockSpec((2, HC), lambda i: (0, 0)),
        pl.BlockSpec((2, HC), lambda i: (0, 0)),
    ],
    out_specs=[
        pl.BlockSpec((BR, HC), lambda i: (i, 0)),
        pl.BlockSpec((4, HC), lambda i: (0, 0)),
    ],
    out_shape=[
        jax.ShapeDtypeStruct((N, HC), jnp.int32),
        jax.ShapeDtypeStruct((4, HC), jnp.float32),
    ],
    scratch_shapes=[
        pltpu.VMEM((2, HC), jnp.float32),
        pltpu.VMEM((2, HC), jnp.float32),
    ],
)


_SC_PARAMS = pltpu.CompilerParams()
if "needs_layout_passes" in pltpu.CompilerParams.__dataclass_fields__:
    _SC_PARAMS = dataclasses.replace(_SC_PARAMS, needs_layout_passes=False)


def _sc_segreduce(x, unq, starts, ab):
    mesh = plsc.VectorSubcoreMesh(core_axis_name="c", subcore_axis_name="s")

    @functools.partial(
        pl.kernel,
        mesh=mesh,
        compiler_params=_SC_PARAMS,
        out_type=(
            jax.ShapeDtypeStruct((NW * SPW * OUT_C,), jnp.float32),
            jax.ShapeDtypeStruct((NW * SPW * OUT_C,), jnp.float32),
        ),
        scratch_types=[
            pltpu.VMEM((48,), jnp.int32),
            pltpu.VMEM((2 * OUT_C,), jnp.float32),
            pltpu.VMEM((CH, HC), jnp.int32),
            pltpu.VMEM((CH, HC), jnp.int32),
            pltpu.VMEM((CH + NLANE,), jnp.int32),
            pltpu.VMEM((CH + NLANE,), jnp.int32),
            pltpu.VMEM((SPW * OUT_C,), jnp.float32),
            pltpu.VMEM((SPW * OUT_C,), jnp.float32),
            pltpu.SemaphoreType.DMA,
            pltpu.SemaphoreType.DMA,
            pltpu.SemaphoreType.DMA,
            pltpu.SemaphoreType.DMA,
        ],
    )
    def body(x_hbm, u_hbm, st_hbm, ab_hbm, omax_hbm, osum_hbm,
             st_v, ab_v, x_v0, x_v1, u_v0, u_v1, smax_v, ssum_v,
             sx0, sx1, su0, su1):
        cc = lax.axis_index("c")
        ss = lax.axis_index("s")
        wid = ss * _INFO.num_cores + cc
        s_lo = wid * SPW

        pltpu.sync_copy(st_hbm, st_v)
        pltpu.sync_copy(ab_hbm, ab_v)
        stv = st_v[pl.ds(wid, NLANE)]
        r_lo = stv[0]
        r_hi = stv[1]

        zeros16 = jnp.zeros((NLANE,), jnp.float32)

        xbufs = (x_v0, x_v1)
        ubufs = (u_v0, u_v1)
        sxs = (sx0, sx1)
        sus = (su0, su1)
        last_base = (N // CH - 1) * CH

        def start(k, b):
            kb = jnp.minimum(k * CH, last_base)
            pltpu.async_copy(
                x_hbm.at[pl.ds(kb, CH)], xbufs[b], sxs[b])
            pltpu.async_copy(
                u_hbm.at[pl.ds(kb, CH)], ubufs[b].at[pl.ds(0, CH)], sus[b])

        def wait(b):
            pltpu.make_async_copy(
                x_hbm.at[pl.ds(0, CH)], xbufs[b], sxs[b]).wait()
            pltpu.make_async_copy(
                u_hbm.at[pl.ds(0, CH)], ubufs[b].at[pl.ds(0, CH)], sus[b]).wait()

        k0 = r_lo // CH
        nk = (r_hi + CH - 1) // CH - k0

        start(k0, 0)

        @pl.loop(0, SPW)
        def _(i):
            ib = i * OUT_C
            for j in range(NVEC):
                smax_v[pl.ds(ib + NLANE * j, NLANE)] = zeros16
                ssum_v[pl.ds(ib + NLANE * j, NLANE)] = zeros16

        a_vecs = [ab_v[pl.ds(NLANE * j, NLANE)] for j in range(NVEC)]
        b_vecs = [ab_v[pl.ds(OUT_C + NLANE * j, NLANE)] for j in range(NVEC)]

        def process(k, b, carry):
            base = k * CH
            lo = jnp.maximum(r_lo - base, 0)
            hi = jnp.minimum(r_hi - base, CH)
            x_v = xbufs[b]
            u_v = ubufs[b]

            def row_body(r, cr):
                cur = cr[0]
                accs = cr[1:1 + NVEC]
                accm = cr[1 + NVEC:]
                seg = u_v[pl.ds(r, NLANE)][0]
                is_new = seg != cur

                @pl.when(is_new & (cur >= 0))
                def _():
                    lb = (cur - s_lo) * OUT_C
                    for j in range(NVEC):
                        ssum_v[pl.ds(lb + NLANE * j, NLANE)] = accs[j]
                        smax_v[pl.ds(lb + NLANE * j, NLANE)] = accm[j]

                keep = jnp.broadcast_to(
                    jnp.where(is_new, 0.0, 1.0).astype(jnp.float32), (NLANE,))
                news = []
                newm = []
                row = x_v.at[r]
                for g in range(NVEC // 2):
                    wi = row[pl.ds(NLANE * g, NLANE)]
                    for j, xv in (
                        (2 * g, plsc.bitcast(wi << 16, jnp.float32)),
                        (2 * g + 1, plsc.bitcast(
                            wi & jnp.int32(-65536), jnp.float32)),
                    ):
                        yv = jnp.maximum(xv * a_vecs[j] + b_vecs[j], 0.0)
                        news.append(accs[j] * keep + yv)
                        newm.append(jnp.maximum(accm[j] * keep, yv))
                return (seg, *news, *newm)

            return lax.fori_loop(lo, hi, row_body, carry)

        def pair_body(i, carry):
            k = k0 + 2 * i
            start(k + 1, 1)
            wait(0)
            carry = process(k, 0, carry)
            start(k + 2, 0)
            wait(1)
            return process(k + 1, 1, carry)

        init = (jnp.int32(-1),) + tuple(zeros16 for _ in range(2 * NVEC))
        fin = lax.fori_loop(0, (nk + 1) // 2, pair_body, init)
        wait(0)
        cur = fin[0]

        @pl.when(cur >= 0)
        def _():
            lb = (cur - s_lo) * OUT_C
            for j in range(NVEC):
                ssum_v[pl.ds(lb + NLANE * j, NLANE)] = fin[1 + j]
                smax_v[pl.ds(lb + NLANE * j, NLANE)] = fin[1 + NVEC + j]

        pltpu.sync_copy(smax_v, omax_hbm.at[pl.ds(s_lo * OUT_C, SPW * OUT_C)])
        pltpu.sync_copy(ssum_v, osum_hbm.at[pl.ds(s_lo * OUT_C, SPW * OUT_C)])

    return body(x, unq, starts, ab)


def _tail_body(xm_ref, gs_ref, w2at_ref, w2bt_ref, g2_ref, b2_ref, o_ref):
    t = (jnp.dot(xm_ref[...], w2at_ref[...], preferred_element_type=jnp.float32)
         + jnp.dot(gs_ref[...], w2bt_ref[...], preferred_element_type=jnp.float32))
    mu = jnp.mean(t, axis=0, keepdims=True)
    d = t - mu
    var = jnp.mean(d * d, axis=0, keepdims=True)
    y = g2_ref[...] * d * lax.rsqrt(var + EPS) + b2_ref[...]
    o_ref[...] = jnp.maximum(y, 0.0)


_phase_c = pl.pallas_call(
    _tail_body,
    out_shape=jax.ShapeDtypeStruct((NUM_SEG, OUT_C), jnp.float32),
)


_PERM = np.concatenate(
    [np.concatenate([np.arange(32 * g, 32 * g + 32, 2),
                     np.arange(32 * g + 1, 32 * g + 32, 2)])
     for g in range(OUT_C // 32)])
_MAP = (_PERM // 2) + HC * (_PERM % 2)


def kernel(inputs, unq_inv, W1, gamma1, beta1, W2, gamma2, beta2):
    g1 = jnp.stack([gamma1[0::2], gamma1[1::2]])
    b1 = jnp.stack([beta1[0::2], beta1[1::2]])
    x, ab = _phase_a(inputs, W1[0::2].T, W1[1::2].T, g1, b1)
    abf = ab.reshape(-1)
    ab_sc = jnp.concatenate([abf[_MAP], abf[2 * HC + _MAP]])

    qs = jnp.arange(0, (NW + 1) * SPW, SPW, dtype=jnp.int32)
    starts = jnp.searchsorted(unq_inv, qs, method="compare_all").astype(jnp.int32)
    starts = jnp.zeros((48,), jnp.int32).at[: NW + 1].set(starts)

    omax, osum = _sc_segreduce(x, unq_inv, starts, ab_sc)
    xm = omax.reshape(NW * SPW, OUT_C)[:NUM_SEG]
    gs = osum.reshape(NW * SPW, OUT_C)[:NUM_SEG]

    w2at = W2[:, 0::2].T[_PERM]
    w2bt = W2[:, 1::2].T[_PERM]
    return _phase_c(xm, gs, w2at, w2bt, gamma2[None], beta2[None])
